# baseline (speedup 1.0000x reference)
import functools

import jax
import jax.numpy as jnp
from jax import lax
from jax.experimental import pallas as pl
from jax.experimental.pallas import tpu as pltpu
from jax.experimental.pallas import tpu_sc as plsc

NC = 2
NS = 16
L = 16
UNROLL = 4

TINY = 1e-6


def _rev(v):
    return lax.rev(v, (0,))


def _bitonic_merge(vs):
    n = len(vs)
    if n == 1:
        return [jnp.sort(vs[0])]
    h = n // 2
    lo = [jnp.minimum(vs[i], vs[i + h]) for i in range(h)]
    hi = [jnp.maximum(vs[i], vs[i + h]) for i in range(h)]
    return _bitonic_merge(lo) + _bitonic_merge(hi)


def _merge_runs(a, b):
    return _bitonic_merge(list(a) + [_rev(x) for x in reversed(b)])


def _sc_body(o_hbm, d_hbm, sv_hbm, w_hbm, u_hbm, pts_hbm, s_hbm,
             ins, outs, sems_in, sems_out, cdf_ss, mid_ss,
             *, rays_per_worker, chunk):
    wid = lax.axis_index("s") * NC + lax.axis_index("c")
    iota = lax.broadcasted_iota(jnp.int32, (L,), 0)
    n_chunks = rays_per_worker // chunk

    lane = [iota + L * k for k in range(4)]
    shift_idx = [jnp.minimum(iota + (L * k + 1), 63) for k in range(4)]
    w_live = [iota + L * k < 62 for k in range(4)]
    cdf_tgt = [jnp.minimum(iota + (L * k + 1), 66) for k in range(4)]
    last = jnp.full((L,), L - 1, jnp.int32)
    zero_f = jnp.zeros((L,), jnp.float32)
    zero_i = jnp.zeros((L,), jnp.int32)
    iota3 = iota * 3

    for cdf_s in cdf_ss:
        cdf_s[pl.ds(0, L)] = zero_f

    def in_pairs(ci, b):
        base = wid * rays_per_worker + ci * chunk
        o_v, d_v, sv_v, w_v, u_v = ins[b]
        sl = pl.ds(base, chunk)
        return [(o_hbm.at[sl], o_v), (d_hbm.at[sl], d_v),
                (sv_hbm.at[sl], sv_v), (w_hbm.at[sl], w_v),
                (u_hbm.at[sl], u_v)]

    def out_pairs(ci, b):
        base = wid * rays_per_worker + ci * chunk
        pts_v, s_v = outs[b]
        sl = pl.ds(base, chunk)
        return [(pts_v, pts_hbm.at[sl]), (s_v, s_hbm.at[sl])]

    def do_ray(r, b, cdf_s, mid_s):
        o_v, d_v, sv_v, w_v, u_v = ins[b]
        pts_v, s_v = outs[b]
        rsp = jnp.full((L,), r, jnp.int32)

        sv = [plsc.load_gather(sv_v, [rsp, lane[k]]) for k in range(4)]
        for k in range(4):
            shifted = plsc.load_gather(sv_v, [rsp, shift_idx[k]])
            mid_s[pl.ds(L * k, L)] = 0.5 * (sv[k] + shifted)

        wc = []
        for k in range(4):
            v = plsc.load_gather(w_v, [rsp, shift_idx[k]]) + TINY
            wc.append(jnp.where(w_live[k], v, 0.0))

        carry = zero_f
        cs = []
        for k in range(4):
            c = plsc.cumsum(wc[k]) + carry
            cs.append(c)
            carry = c[last]
        inv = 1.0 / carry
        for k in range(4):
            plsc.store_scatter(cdf_s, [cdf_tgt[k]], cs[k] * inv)

        sn = []
        for k in range(4):
            uk = plsc.load_gather(u_v, [rsp, lane[k]])
            pos = zero_i
            for sz in (32, 16, 8, 4):
                cand = pos + sz
                val = plsc.load_gather(cdf_s, [cand])
                pos = jnp.where(val <= uk, cand, pos)
            for sz in (2, 1):
                cand = pos + sz
                val = plsc.load_gather(cdf_s, [jnp.minimum(cand, 61)])
                pos = jnp.where((val <= uk) & (cand <= 61), cand, pos)
            cb = plsc.load_gather(cdf_s, [pos])
            ca = plsc.load_gather(cdf_s, [pos + 1])
            bb = plsc.load_gather(mid_s, [pos])
            ba = plsc.load_gather(mid_s, [pos + 1])
            denom = ca - cb
            denom = jnp.where(denom < TINY, 1.0, denom)
            t = (uk - cb) / denom
            sn.append(bb + t * (ba - bb + TINY))

        sn = [jnp.sort(x) for x in sn]
        r01 = _merge_runs(sn[0:1], sn[1:2])
        r23 = _merge_runs(sn[2:3], sn[3:4])
        run4 = _merge_runs(r01, r23)
        s8 = _merge_runs(run4, sv)

        for j in range(8):
            plsc.store_scatter(s_v, [rsp, lane[j % 4] + L * (j - j % 4)],
                               s8[j])

        od = []
        for c in range(3):
            csp = jnp.full((L,), c, jnp.int32)
            od.append((plsc.load_gather(o_v, [rsp, csp]),
                       plsc.load_gather(d_v, [rsp, csp])))
        for j in range(8):
            colbase = iota3 + L * 3 * j
            for c in range(3):
                oc, dc = od[c]
                plsc.store_scatter(pts_v, [rsp, colbase + c],
                                   oc + dc * s8[j])

    def compute_chunk(b):
        def ray_group(i, carry_unused):
            for s in range(UNROLL):
                do_ray(i * UNROLL + s, b, cdf_ss[s], mid_ss[s])
            return carry_unused
        lax.fori_loop(0, chunk // UNROLL, ray_group, 0)

    for src, dst in in_pairs(0, 0):
        pltpu.async_copy(src, dst, sems_in[0])

    def grp(g, carry_unused):
        for b in range(2):
            ci = 2 * g + b
            for src, dst in in_pairs(ci, b):
                pltpu.make_async_copy(src, dst, sems_in[b]).wait()

            @pl.when(ci + 1 < n_chunks)
            def _start_next():
                for src, dst in in_pairs(ci + 1, 1 - b):
                    pltpu.async_copy(src, dst, sems_in[1 - b])

            @pl.when(ci >= 2)
            def _drain_prev():
                for src, dst in out_pairs(ci - 2, b):
                    pltpu.make_async_copy(src, dst, sems_out[b]).wait()

            compute_chunk(b)
            for src, dst in out_pairs(ci, b):
                pltpu.async_copy(src, dst, sems_out[b])
        return carry_unused

    lax.fori_loop(0, n_chunks // 2, grp, 0)
    for src, dst in out_pairs(n_chunks - 2, 0):
        pltpu.make_async_copy(src, dst, sems_out[0]).wait()
    for src, dst in out_pairs(n_chunks - 1, 1):
        pltpu.make_async_copy(src, dst, sems_out[1]).wait()


def kernel(rays_o, rays_d, s_vals, weights, u):
    B, M = s_vals.shape
    assert M == 64 and u.shape[-1] == 64
    nw = NC * NS
    rays_per_worker = B // nw
    chunk = 32
    assert rays_per_worker % (2 * chunk) == 0

    body = functools.partial(_sc_body, rays_per_worker=rays_per_worker,
                             chunk=chunk)
    def in_set():
        return [
            pltpu.VMEM((chunk, 3), jnp.float32),
            pltpu.VMEM((chunk, 3), jnp.float32),
            pltpu.VMEM((chunk, 64), jnp.float32),
            pltpu.VMEM((chunk, 64), jnp.float32),
            pltpu.VMEM((chunk, 64), jnp.float32),
        ]
    def out_set():
        return [
            pltpu.VMEM((chunk, 384), jnp.float32),
            pltpu.VMEM((chunk, 128), jnp.float32),
        ]
    f = pl.kernel(
        body,
        out_type=[
            jax.ShapeDtypeStruct((B, 384), jnp.float32),
            jax.ShapeDtypeStruct((B, 128), jnp.float32),
        ],
        mesh=plsc.VectorSubcoreMesh(
            core_axis_name="c", subcore_axis_name="s",
            num_cores=NC, num_subcores=NS),
        compiler_params=pltpu.CompilerParams(
            needs_layout_passes=False, disable_bounds_checks=True),
        scratch_types=[
            [in_set(), in_set()],
            [out_set(), out_set()],
            [pltpu.SemaphoreType.DMA] * 2,
            [pltpu.SemaphoreType.DMA] * 2,
            [pltpu.VMEM((80,), jnp.float32)] * UNROLL,
            [pltpu.VMEM((80,), jnp.float32)] * UNROLL,
        ],
    )
    pts2d, s = f(rays_o, rays_d, s_vals, weights, u)
    pts = pts2d.reshape(B, 128, 3)
    return (pts, s, s)


# speedup vs baseline: 1.0089x; 1.0089x over previous
import functools

import jax
import jax.numpy as jnp
from jax import lax
from jax.experimental import pallas as pl
from jax.experimental.pallas import tpu as pltpu
from jax.experimental.pallas import tpu_sc as plsc

NC = 2
NS = 16
L = 16
UNROLL = 2

TINY = 1e-6


def _rev(v):
    return lax.rev(v, (0,))


def _bitonic_merge(vs):
    n = len(vs)
    if n == 1:
        return [jnp.sort(vs[0])]
    h = n // 2
    lo = [jnp.minimum(vs[i], vs[i + h]) for i in range(h)]
    hi = [jnp.maximum(vs[i], vs[i + h]) for i in range(h)]
    return _bitonic_merge(lo) + _bitonic_merge(hi)


def _merge_runs(a, b):
    return _bitonic_merge(list(a) + [_rev(x) for x in reversed(b)])


def _sc_body(o_hbm, d_hbm, sv_hbm, w_hbm, u_hbm, pts_hbm, s_hbm,
             ins, outs, sems_in, sems_out, cdf_ss, mid_ss,
             *, rays_per_worker, chunk):
    wid = lax.axis_index("s") * NC + lax.axis_index("c")
    iota = lax.broadcasted_iota(jnp.int32, (L,), 0)
    n_chunks = rays_per_worker // chunk

    lane = [iota + L * k for k in range(4)]
    shift_idx = [jnp.minimum(iota + (L * k + 1), 63) for k in range(4)]
    w_live = [iota + L * k < 62 for k in range(4)]
    cdf_tgt = [jnp.minimum(iota + (L * k + 1), 66) for k in range(4)]
    last = jnp.full((L,), L - 1, jnp.int32)
    zero_f = jnp.zeros((L,), jnp.float32)
    zero_i = jnp.zeros((L,), jnp.int32)
    iota3 = iota * 3

    for cdf_s in cdf_ss:
        cdf_s[pl.ds(0, L)] = zero_f

    def in_pairs(ci, b):
        base = wid * rays_per_worker + ci * chunk
        o_v, d_v, sv_v, w_v, u_v = ins[b]
        sl = pl.ds(base, chunk)
        return [(o_hbm.at[sl], o_v), (d_hbm.at[sl], d_v),
                (sv_hbm.at[sl], sv_v), (w_hbm.at[sl], w_v),
                (u_hbm.at[sl], u_v)]

    def out_pairs(ci, b):
        base = wid * rays_per_worker + ci * chunk
        pts_v, s_v = outs[b]
        sl = pl.ds(base, chunk)
        return [(pts_v, pts_hbm.at[sl]), (s_v, s_hbm.at[sl])]

    def do_ray(r, b, cdf_s, mid_s):
        o_v, d_v, sv_v, w_v, u_v = ins[b]
        pts_v, s_v = outs[b]
        rsp = jnp.full((L,), r, jnp.int32)

        sv = [plsc.load_gather(sv_v, [rsp, lane[k]]) for k in range(4)]
        for k in range(4):
            shifted = plsc.load_gather(sv_v, [rsp, shift_idx[k]])
            mid_s[pl.ds(L * k, L)] = 0.5 * (sv[k] + shifted)

        wc = []
        for k in range(4):
            v = plsc.load_gather(w_v, [rsp, shift_idx[k]]) + TINY
            wc.append(jnp.where(w_live[k], v, 0.0))

        carry = zero_f
        cs = []
        for k in range(4):
            c = plsc.cumsum(wc[k]) + carry
            cs.append(c)
            carry = c[last]
        inv = 1.0 / carry
        for k in range(4):
            plsc.store_scatter(cdf_s, [cdf_tgt[k]], cs[k] * inv)

        sn = []
        for k in range(4):
            uk = plsc.load_gather(u_v, [rsp, lane[k]])
            pos = zero_i
            for sz in (32, 16, 8, 4):
                cand = pos + sz
                val = plsc.load_gather(cdf_s, [cand])
                pos = jnp.where(val <= uk, cand, pos)
            for sz in (2, 1):
                cand = pos + sz
                val = plsc.load_gather(cdf_s, [jnp.minimum(cand, 61)])
                pos = jnp.where((val <= uk) & (cand <= 61), cand, pos)
            cb = plsc.load_gather(cdf_s, [pos])
            ca = plsc.load_gather(cdf_s, [pos + 1])
            bb = plsc.load_gather(mid_s, [pos])
            ba = plsc.load_gather(mid_s, [pos + 1])
            denom = ca - cb
            denom = jnp.where(denom < TINY, 1.0, denom)
            t = (uk - cb) / denom
            sn.append(bb + t * (ba - bb + TINY))

        sn = [jnp.sort(x) for x in sn]
        r01 = _merge_runs(sn[0:1], sn[1:2])
        r23 = _merge_runs(sn[2:3], sn[3:4])
        run4 = _merge_runs(r01, r23)
        s8 = _merge_runs(run4, sv)

        for j in range(8):
            plsc.store_scatter(s_v, [rsp, lane[j % 4] + L * (j - j % 4)],
                               s8[j])

        od = []
        for c in range(3):
            csp = jnp.full((L,), c, jnp.int32)
            od.append((plsc.load_gather(o_v, [rsp, csp]),
                       plsc.load_gather(d_v, [rsp, csp])))
        for j in range(8):
            colbase = iota3 + L * 3 * j
            for c in range(3):
                oc, dc = od[c]
                plsc.store_scatter(pts_v, [rsp, colbase + c],
                                   oc + dc * s8[j])

    def compute_chunk(b):
        def ray_group(i, carry_unused):
            for s in range(UNROLL):
                do_ray(i * UNROLL + s, b, cdf_ss[s], mid_ss[s])
            return carry_unused
        lax.fori_loop(0, chunk // UNROLL, ray_group, 0)

    for src, dst in in_pairs(0, 0):
        pltpu.async_copy(src, dst, sems_in[0])

    def grp(g, carry_unused):
        for b in range(2):
            ci = 2 * g + b
            for src, dst in in_pairs(ci, b):
                pltpu.make_async_copy(src, dst, sems_in[b]).wait()

            @pl.when(ci + 1 < n_chunks)
            def _start_next():
                for src, dst in in_pairs(ci + 1, 1 - b):
                    pltpu.async_copy(src, dst, sems_in[1 - b])

            @pl.when(ci >= 2)
            def _drain_prev():
                for src, dst in out_pairs(ci - 2, b):
                    pltpu.make_async_copy(src, dst, sems_out[b]).wait()

            compute_chunk(b)
            for src, dst in out_pairs(ci, b):
                pltpu.async_copy(src, dst, sems_out[b])
        return carry_unused

    lax.fori_loop(0, n_chunks // 2, grp, 0)
    for src, dst in out_pairs(n_chunks - 2, 0):
        pltpu.make_async_copy(src, dst, sems_out[0]).wait()
    for src, dst in out_pairs(n_chunks - 1, 1):
        pltpu.make_async_copy(src, dst, sems_out[1]).wait()


def kernel(rays_o, rays_d, s_vals, weights, u):
    B, M = s_vals.shape
    assert M == 64 and u.shape[-1] == 64
    nw = NC * NS
    rays_per_worker = B // nw
    chunk = 32
    assert rays_per_worker % (2 * chunk) == 0

    body = functools.partial(_sc_body, rays_per_worker=rays_per_worker,
                             chunk=chunk)
    def in_set():
        return [
            pltpu.VMEM((chunk, 3), jnp.float32),
            pltpu.VMEM((chunk, 3), jnp.float32),
            pltpu.VMEM((chunk, 64), jnp.float32),
            pltpu.VMEM((chunk, 64), jnp.float32),
            pltpu.VMEM((chunk, 64), jnp.float32),
        ]
    def out_set():
        return [
            pltpu.VMEM((chunk, 384), jnp.float32),
            pltpu.VMEM((chunk, 128), jnp.float32),
        ]
    f = pl.kernel(
        body,
        out_type=[
            jax.ShapeDtypeStruct((B, 384), jnp.float32),
            jax.ShapeDtypeStruct((B, 128), jnp.float32),
        ],
        mesh=plsc.VectorSubcoreMesh(
            core_axis_name="c", subcore_axis_name="s",
            num_cores=NC, num_subcores=NS),
        compiler_params=pltpu.CompilerParams(
            needs_layout_passes=False, disable_bounds_checks=True),
        scratch_types=[
            [in_set(), in_set()],
            [out_set(), out_set()],
            [pltpu.SemaphoreType.DMA] * 2,
            [pltpu.SemaphoreType.DMA] * 2,
            [pltpu.VMEM((80,), jnp.float32)] * UNROLL,
            [pltpu.VMEM((80,), jnp.float32)] * UNROLL,
        ],
    )
    pts2d, s = f(rays_o, rays_d, s_vals, weights, u)
    pts = pts2d.reshape(B, 128, 3)
    return (pts, s, s)
